# Initial kernel scaffold; baseline (speedup 1.0000x reference)
import jax
import jax.numpy as jnp
from jax.experimental import pallas as pl


def kernel(x, edge_index, W, b, eps):
    raise NotImplementedError("write your pallas kernel here")


# speedup vs baseline: 1.2317x; 1.2317x over previous
import functools

import jax
import jax.numpy as jnp
from jax import lax
from jax.experimental import pallas as pl
from jax.experimental.pallas import tpu as pltpu
from jax.experimental.pallas import tpu_sc as plsc

N = 10000
K = 32
D = 128
NW = 32
CK = 128
C = CK // K
NPAD = 10240
ROWS_W = NPAD // NW
CHUNKS = ROWS_W // C
NV = D // 16


def _sc_mesh():
    return plsc.VectorSubcoreMesh(core_axis_name="c", subcore_axis_name="s")


@functools.partial(
    pl.kernel,
    mesh=_sc_mesh(),
    out_type=jax.ShapeDtypeStruct((NW, ROWS_W, D), jnp.float32),
    scratch_types=[
        pltpu.VMEM((CHUNKS, CK), jnp.int32),
        pltpu.VMEM((CK, D), jnp.float32),
        pltpu.VMEM((C, D), jnp.float32),
        pltpu.SemaphoreType.DMA,
    ],
)
def _sc_gather_sum(x_hbm, ei_hbm, s_hbm, idx_v, rows_v, out_v, sem):
    wid = lax.axis_index("s") * 2 + lax.axis_index("c")
    pltpu.sync_copy(ei_hbm.at[wid], idx_v)

    def chunk_body(t, carry):
        pltpu.async_copy(x_hbm.at[idx_v.at[t]], rows_v, sem).wait()
        for c in range(C):
            def kbody(k, accs):
                r = c * K + k
                return tuple(accs[v] + rows_v[r, pl.ds(v * 16, 16)]
                             for v in range(NV))
            accs = tuple(jnp.zeros((16,), jnp.float32) for _ in range(NV))
            accs = lax.fori_loop(0, K, kbody, accs)
            for v in range(NV):
                out_v[c, pl.ds(v * 16, 16)] = accs[v]
        pltpu.sync_copy(out_v, s_hbm.at[wid, pl.ds(t * C, C)])
        return carry

    lax.fori_loop(0, CHUNKS, chunk_body, 0)


def _tc_body(eps_ref, x_ref, s_ref, w_ref, b_ref, o_ref):
    h = (1.0 + eps_ref[0, 0]) * x_ref[...] + s_ref[...]
    o_ref[...] = lax.dot_general(
        h, w_ref[...], (((1,), (1,)), ((), ())),
        preferred_element_type=jnp.float32) + b_ref[...]


_TC_BLK = 2000


def _tc_mlp(eps, x, s, W, b):
    grid = (N // _TC_BLK,)
    return pl.pallas_call(
        _tc_body,
        grid=grid,
        in_specs=[
            pl.BlockSpec(memory_space=pltpu.SMEM),
            pl.BlockSpec((_TC_BLK, D), lambda i: (i, 0)),
            pl.BlockSpec((_TC_BLK, D), lambda i: (i, 0)),
            pl.BlockSpec((D, D), lambda i: (0, 0)),
            pl.BlockSpec((1, D), lambda i: (0, 0)),
        ],
        out_specs=pl.BlockSpec((_TC_BLK, D), lambda i: (i, 0)),
        out_shape=jax.ShapeDtypeStruct((N, D), jnp.float32),
    )(eps, x, s, W, b)


def kernel(x, edge_index, W, b, eps):
    ei_flat = edge_index.reshape(-1)
    pad = jnp.zeros((NPAD * K - N * K,), jnp.int32)
    ei3 = jnp.concatenate([ei_flat, pad]).reshape(NW, CHUNKS, CK)
    s = _sc_gather_sum(x, ei3).reshape(NPAD, D)
    eps2 = eps.reshape(1, 1)
    b2 = b.reshape(1, D)
    return _tc_mlp(eps2, x, s, W, b2)


# speedup vs baseline: 1.3651x; 1.1083x over previous
import functools

import jax
import jax.numpy as jnp
from jax import lax
from jax.experimental import pallas as pl
from jax.experimental.pallas import tpu as pltpu
from jax.experimental.pallas import tpu_sc as plsc

N = 10000
K = 32
D = 128
NW = 32
CK = 128
G = 2
C = G * CK // K
NPAD = 10240
ROWS_W = NPAD // NW
CHUNKS = ROWS_W // C
GCH = CHUNKS * G
NV = D // 16


def _sc_mesh():
    return plsc.VectorSubcoreMesh(core_axis_name="c", subcore_axis_name="s")


@functools.partial(
    pl.kernel,
    mesh=_sc_mesh(),
    out_type=jax.ShapeDtypeStruct((NW, ROWS_W, D), jnp.float32),
    scratch_types=[
        pltpu.VMEM((GCH, CK), jnp.int32),
        pltpu.VMEM((G * CK, D), jnp.float32),
        pltpu.VMEM((G * CK, D), jnp.float32),
        pltpu.VMEM((C, D), jnp.float32),
        pltpu.SemaphoreType.DMA,
        pltpu.SemaphoreType.DMA,
    ],
)
def _sc_gather_sum(x_hbm, ei_hbm, s_hbm, idx_v, buf_a, buf_b, out_v,
                   sem_a, sem_b):
    wid = lax.axis_index("s") * 2 + lax.axis_index("c")
    pltpu.sync_copy(ei_hbm.at[wid], idx_v)

    def fire(t, buf, sem):
        for g in range(G):
            pltpu.async_copy(x_hbm.at[idx_v.at[t * G + g]],
                             buf.at[pl.ds(g * CK, CK)], sem)

    def drain(t, buf, sem):
        for g in range(G):
            pltpu.make_async_copy(x_hbm.at[idx_v.at[t * G + g]],
                                  buf.at[pl.ds(g * CK, CK)], sem).wait()

    def acc_chunk(t, buf):
        def cbody(c, carry):
            def kbody(kk, accs):
                r0 = c * K + kk * 8
                for j in range(8):
                    accs = tuple(accs[v] + buf[r0 + j, pl.ds(v * 16, 16)]
                                 for v in range(NV))
                return accs
            accs = tuple(jnp.zeros((16,), jnp.float32) for _ in range(NV))
            accs = lax.fori_loop(0, K // 8, kbody, accs)
            for v in range(NV):
                out_v[c, pl.ds(v * 16, 16)] = accs[v]
            return carry
        lax.fori_loop(0, C, cbody, 0)
        pltpu.sync_copy(out_v, s_hbm.at[wid, pl.ds(t * C, C)])

    fire(0, buf_a, sem_a)

    def body(u, carry):
        t0 = 2 * u
        t1 = t0 + 1
        fire(t1, buf_b, sem_b)
        drain(t0, buf_a, sem_a)
        acc_chunk(t0, buf_a)

        @pl.when(t0 + 2 < CHUNKS)
        def _():
            fire(t0 + 2, buf_a, sem_a)

        drain(t1, buf_b, sem_b)
        acc_chunk(t1, buf_b)
        return carry

    lax.fori_loop(0, CHUNKS // 2, body, 0)


def _tc_body(eps_ref, x_ref, s_ref, w_ref, b_ref, o_ref):
    h = (1.0 + eps_ref[0, 0]) * x_ref[...] + s_ref[...]
    o_ref[...] = lax.dot_general(
        h, w_ref[...], (((1,), (1,)), ((), ())),
        preferred_element_type=jnp.float32) + b_ref[...]


_TC_BLK = 2000


def _tc_mlp(eps, x, s, W, b):
    grid = (N // _TC_BLK,)
    return pl.pallas_call(
        _tc_body,
        grid=grid,
        in_specs=[
            pl.BlockSpec(memory_space=pltpu.SMEM),
            pl.BlockSpec((_TC_BLK, D), lambda i: (i, 0)),
            pl.BlockSpec((_TC_BLK, D), lambda i: (i, 0)),
            pl.BlockSpec((D, D), lambda i: (0, 0)),
            pl.BlockSpec((1, D), lambda i: (0, 0)),
        ],
        out_specs=pl.BlockSpec((_TC_BLK, D), lambda i: (i, 0)),
        out_shape=jax.ShapeDtypeStruct((N, D), jnp.float32),
    )(eps, x, s, W, b)


def kernel(x, edge_index, W, b, eps):
    ei_flat = edge_index.reshape(-1)
    pad = jnp.zeros((NPAD * K - N * K,), jnp.int32)
    ei3 = jnp.concatenate([ei_flat, pad]).reshape(NW, GCH, CK)
    s = _sc_gather_sum(x, ei3).reshape(NPAD, D)
    eps2 = eps.reshape(1, 1)
    b2 = b.reshape(1, D)
    return _tc_mlp(eps2, x, s, W, b2)


# speedup vs baseline: 6.6770x; 4.8913x over previous
import functools

import jax
import jax.numpy as jnp
from jax import lax
from jax.experimental import pallas as pl
from jax.experimental.pallas import tpu as pltpu
from jax.experimental.pallas import tpu_sc as plsc

N = 10000
K = 32
D = 128
NW = 32
CK = 64
C = CK // K
NPAD = 10240
ROWS_W = NPAD // NW
CHUNKS = ROWS_W // C
GCH = CHUNKS
SE = 4
NV = D // 16


def _sc_mesh():
    return plsc.VectorSubcoreMesh(core_axis_name="c", subcore_axis_name="s")


SH_PER_SUB = NPAD // 16


@functools.partial(
    pl.kernel,
    mesh=_sc_mesh(),
    out_type=jax.ShapeDtypeStruct((NW, ROWS_W, D), jnp.float32),
    scratch_types=[
        pltpu.VMEM((GCH, CK), jnp.int32),
        pltpu.VMEM((CK, D), jnp.float32),
        pltpu.VMEM((CK, D), jnp.float32),
        pltpu.VMEM((SE * C, D), jnp.float32),
        pltpu.VMEM_SHARED((NPAD, D), jnp.float32),
        pltpu.SemaphoreType.DMA,
        pltpu.SemaphoreType.DMA,
    ],
)
def _sc_gather_sum(x_hbm, ei_hbm, s_hbm, idx_v, buf_a, buf_b, out_v,
                   x_sh, sem_a, sem_b):
    sub = lax.axis_index("s")
    wid = sub * 2 + lax.axis_index("c")
    pltpu.sync_copy(x_hbm.at[pl.ds(sub * SH_PER_SUB, SH_PER_SUB)],
                    x_sh.at[pl.ds(sub * SH_PER_SUB, SH_PER_SUB)])
    pltpu.sync_copy(ei_hbm.at[wid], idx_v)
    plsc.subcore_barrier()

    def fire(t, buf, sem):
        pltpu.async_copy(x_sh.at[idx_v.at[t]], buf, sem)

    def drain(t, buf, sem):
        pltpu.make_async_copy(x_sh.at[idx_v.at[t]], buf, sem).wait()

    def acc_chunk(t, buf):
        ob = (t % SE) * C
        def cbody(c, carry):
            def kbody(kk, accs):
                r0 = c * K + kk * 8
                for j in range(8):
                    accs = tuple(accs[v] + buf[r0 + j, pl.ds(v * 16, 16)]
                                 for v in range(NV))
                return accs
            accs = tuple(jnp.zeros((16,), jnp.float32) for _ in range(NV))
            accs = lax.fori_loop(0, K // 8, kbody, accs)
            for v in range(NV):
                out_v[ob + c, pl.ds(v * 16, 16)] = accs[v]
            return carry
        lax.fori_loop(0, C, cbody, 0)

        @pl.when(t % SE == SE - 1)
        def _():
            base = pl.multiple_of((t - (SE - 1)) * C, SE * C)
            pltpu.sync_copy(out_v, s_hbm.at[wid, pl.ds(base, SE * C)])

    fire(0, buf_a, sem_a)

    def body(u, carry):
        t0 = 2 * u
        t1 = t0 + 1
        fire(t1, buf_b, sem_b)
        drain(t0, buf_a, sem_a)
        acc_chunk(t0, buf_a)

        @pl.when(t0 + 2 < CHUNKS)
        def _():
            fire(t0 + 2, buf_a, sem_a)

        drain(t1, buf_b, sem_b)
        acc_chunk(t1, buf_b)
        return carry

    lax.fori_loop(0, CHUNKS // 2, body, 0)


def _tc_body(eps_ref, x_ref, s_ref, w_ref, b_ref, o_ref):
    h = (1.0 + eps_ref[0, 0]) * x_ref[...] + s_ref[...]
    o_ref[...] = lax.dot_general(
        h, w_ref[...], (((1,), (1,)), ((), ())),
        preferred_element_type=jnp.float32) + b_ref[...]


_TC_BLK = 2000


def _tc_mlp(eps, x, s, W, b):
    grid = (N // _TC_BLK,)
    return pl.pallas_call(
        _tc_body,
        grid=grid,
        in_specs=[
            pl.BlockSpec(memory_space=pltpu.SMEM),
            pl.BlockSpec((_TC_BLK, D), lambda i: (i, 0)),
            pl.BlockSpec((_TC_BLK, D), lambda i: (i, 0)),
            pl.BlockSpec((D, D), lambda i: (0, 0)),
            pl.BlockSpec((1, D), lambda i: (0, 0)),
        ],
        out_specs=pl.BlockSpec((_TC_BLK, D), lambda i: (i, 0)),
        out_shape=jax.ShapeDtypeStruct((N, D), jnp.float32),
    )(eps, x, s, W, b)


def kernel(x, edge_index, W, b, eps):
    ei_flat = edge_index.reshape(-1)
    pad = jnp.zeros((NPAD * K - N * K,), jnp.int32)
    ei3 = jnp.concatenate([ei_flat, pad]).reshape(NW, GCH, CK)
    x_pad = jnp.concatenate(
        [x, jnp.zeros((NPAD - N, D), jnp.float32)], axis=0)
    s = _sc_gather_sum(x_pad, ei3).reshape(NPAD, D)
    eps2 = eps.reshape(1, 1)
    b2 = b.reshape(1, D)
    return _tc_mlp(eps2, x, s, W, b2)


# speedup vs baseline: 7.5177x; 1.1259x over previous
import functools

import jax
import jax.numpy as jnp
from jax import lax
from jax.experimental import pallas as pl
from jax.experimental.pallas import tpu as pltpu
from jax.experimental.pallas import tpu_sc as plsc

N = 10000
K = 32
D = 128
NW = 32
CK = 64
C = CK // K
ROWS_W = 320
CHUNKS = ROWS_W // C
SE = 4
NV = D // 16
SH_PER_SUB = 640


def _sc_mesh():
    return plsc.VectorSubcoreMesh(core_axis_name="c", subcore_axis_name="s")


@functools.partial(
    pl.kernel,
    mesh=_sc_mesh(),
    out_type=jax.ShapeDtypeStruct((N, D), jnp.float32),
    scratch_types=[
        pltpu.VMEM((ROWS_W * K,), jnp.int32),
        pltpu.VMEM((CK, D), jnp.float32),
        pltpu.VMEM((CK, D), jnp.float32),
        pltpu.VMEM((SE * C, D), jnp.float32),
        pltpu.VMEM_SHARED((N, D), jnp.float32),
        pltpu.SemaphoreType.DMA,
        pltpu.SemaphoreType.DMA,
    ],
)
def _sc_gather_sum(x_hbm, ei_hbm, s_hbm, idx_v, buf_a, buf_b, out_v,
                   x_sh, sem_a, sem_b):
    sub = lax.axis_index("s")
    wid = sub * 2 + lax.axis_index("c")
    xoff = jnp.minimum(sub * SH_PER_SUB, N - SH_PER_SUB)
    xoff = pl.multiple_of(xoff, 16)
    pltpu.sync_copy(x_hbm.at[pl.ds(xoff, SH_PER_SUB)],
                    x_sh.at[pl.ds(xoff, SH_PER_SUB)])
    woff = jnp.minimum(wid * ROWS_W, N - ROWS_W)
    woff = pl.multiple_of(woff, 16)
    pltpu.sync_copy(ei_hbm.at[pl.ds(woff * K, ROWS_W * K)], idx_v)
    plsc.subcore_barrier()

    def fire(t, buf, sem):
        pltpu.async_copy(x_sh.at[idx_v.at[pl.ds(t * CK, CK)]], buf, sem)

    def drain(t, buf, sem):
        pltpu.make_async_copy(x_sh.at[idx_v.at[pl.ds(t * CK, CK)]],
                              buf, sem).wait()

    def acc_chunk(t, buf):
        ob = (t % SE) * C

        def cbody(c, carry):
            def kbody(kk, accs):
                r0 = c * K + kk * 8
                for j in range(8):
                    accs = tuple(accs[v] + buf[r0 + j, pl.ds(v * 16, 16)]
                                 for v in range(NV))
                return accs
            accs = tuple(jnp.zeros((16,), jnp.float32) for _ in range(NV))
            accs = lax.fori_loop(0, K // 8, kbody, accs)
            for v in range(NV):
                out_v[ob + c, pl.ds(v * 16, 16)] = accs[v]
            return carry
        lax.fori_loop(0, C, cbody, 0)

        @pl.when(t % SE == SE - 1)
        def _():
            base = pl.multiple_of(woff + (t - (SE - 1)) * C, SE * C)
            pltpu.sync_copy(out_v, s_hbm.at[pl.ds(base, SE * C)])

    fire(0, buf_a, sem_a)

    def body(u, carry):
        t0 = 2 * u
        t1 = t0 + 1
        fire(t1, buf_b, sem_b)
        drain(t0, buf_a, sem_a)
        acc_chunk(t0, buf_a)

        @pl.when(t0 + 2 < CHUNKS)
        def _():
            fire(t0 + 2, buf_a, sem_a)

        drain(t1, buf_b, sem_b)
        acc_chunk(t1, buf_b)
        return carry

    lax.fori_loop(0, CHUNKS // 2, body, 0)


def _tc_body(eps_ref, x_ref, s_ref, w_ref, b_ref, o_ref):
    h = (1.0 + eps_ref[0, 0]) * x_ref[...] + s_ref[...]
    o_ref[...] = lax.dot_general(
        h, w_ref[...], (((1,), (1,)), ((), ())),
        preferred_element_type=jnp.float32) + b_ref[...]


_TC_BLK = 2000


def _tc_mlp(eps, x, s, W, b):
    grid = (N // _TC_BLK,)
    return pl.pallas_call(
        _tc_body,
        grid=grid,
        in_specs=[
            pl.BlockSpec(memory_space=pltpu.SMEM),
            pl.BlockSpec((_TC_BLK, D), lambda i: (i, 0)),
            pl.BlockSpec((_TC_BLK, D), lambda i: (i, 0)),
            pl.BlockSpec((D, D), lambda i: (0, 0)),
            pl.BlockSpec((1, D), lambda i: (0, 0)),
        ],
        out_specs=pl.BlockSpec((_TC_BLK, D), lambda i: (i, 0)),
        out_shape=jax.ShapeDtypeStruct((N, D), jnp.float32),
    )(eps, x, s, W, b)


def kernel(x, edge_index, W, b, eps):
    ei1d = edge_index.reshape(-1)
    s = _sc_gather_sum(x, ei1d)
    eps2 = eps.reshape(1, 1)
    b2 = b.reshape(1, D)
    return _tc_mlp(eps2, x, s, W, b2)
